# baseline (speedup 1.0000x reference)
import functools

import jax
import jax.numpy as jnp
from jax import lax
from jax.experimental import pallas as pl
from jax.experimental.pallas import tpu as pltpu
from jax.experimental.pallas import tpu_sc as plsc

NC = 2
NS = 16
L = 16
NW = NC * NS

D = 128
CHUNK = 128
NBC = 8
SC_SLOW_CORE = 1
SC_SLOW_FRAC_NUM = 26
MB = 1280
N_PAD = 10240
RPS = N_PAD // NS


def _mesh():
    return plsc.VectorSubcoreMesh(
        core_axis_name="c", subcore_axis_name="s", num_cores=NC, num_subcores=NS
    )


@functools.lru_cache(maxsize=None)
def _sc_degree(e_pad):
    ept = e_pad // NW
    nch = ept // CHUNK

    @functools.partial(
        pl.kernel,
        out_type=jax.ShapeDtypeStruct((NC * N_PAD,), jnp.float32),
        mesh=_mesh(),
        scratch_types=[
            pltpu.VMEM((ept // CHUNK, CHUNK), jnp.int32),
            pltpu.VMEM((CHUNK,), jnp.float32),
            pltpu.VMEM((RPS,), jnp.float32),
            pltpu.VMEM_SHARED((N_PAD,), jnp.float32),
            pltpu.SemaphoreType.DMA,
            pltpu.SemaphoreType.DMA,
            pltpu.SemaphoreType.DMA,
            pltpu.SemaphoreType.DMA,
        ],
    )
    def deg_kernel(row_hbm, out_hbm, idx_a, ones_v, zero_v, s1, sm0, sm1, sm2, sm3):
        c = lax.axis_index("c")
        s = lax.axis_index("s")
        wid = s * NC + c
        sems = (sm0, sm1, sm2, sm3)
        ones = jnp.ones((L,), jnp.float32)
        zeros = jnp.zeros((L,), jnp.float32)
        for j in range(CHUNK // L):
            ones_v[pl.ds(j * L, L)] = ones

        def zfill(i, _):
            zero_v[pl.ds(i * L, L)] = zeros
            return 0

        lax.fori_loop(0, RPS // L, zfill, 0)
        pltpu.sync_copy(zero_v, s1.at[pl.ds(s * RPS, RPS)])
        pltpu.sync_copy(row_hbm.at[pl.ds(wid * nch, nch)], idx_a)
        plsc.subcore_barrier()

        def quad(i, _):
            for b in range(4):
                j = 4 * i + b
                pltpu.make_async_copy(ones_v, s1.at[idx_a.at[j]], sems[b]).wait()
                pltpu.async_copy(ones_v, s1.at[idx_a.at[j + 4]], sems[b], add=True)
            return 0

        for b in range(4):
            pltpu.async_copy(ones_v, s1.at[idx_a.at[b]], sems[b], add=True)
        lax.fori_loop(0, nch // 4 - 1, quad, 0)
        for b in range(4):
            pltpu.make_async_copy(ones_v, s1.at[idx_a.at[0]], sems[b]).wait()
        plsc.subcore_barrier()
        pltpu.sync_copy(
            s1.at[pl.ds(s * RPS, RPS)],
            out_hbm.at[pl.ds(c * N_PAD + s * RPS, RPS)],
        )

    return deg_kernel


@functools.lru_cache(maxsize=None)
def _sc_segsum(e_pad):
    totch = e_pad // CHUNK
    a_ch = (totch * SC_SLOW_FRAC_NUM // 128) // (NS * NBC) * (NS * NBC)
    b_ch = totch - a_ch
    nch0 = a_ch // NS
    nch1 = b_ch // NS

    @functools.partial(
        pl.kernel,
        out_type=jax.ShapeDtypeStruct((NC, N_PAD, D), jnp.float32),
        mesh=_mesh(),
        scratch_types=[
            pltpu.VMEM((2, NBC, CHUNK), jnp.int32),
            pltpu.VMEM((2, NBC, CHUNK), jnp.int32),
            pltpu.VMEM((2, CHUNK, D), jnp.float32),
            pltpu.VMEM_SHARED((N_PAD, D), jnp.float32),
            pltpu.SemaphoreType.DMA,
            pltpu.SemaphoreType.DMA,
            pltpu.SemaphoreType.DMA,
            pltpu.SemaphoreType.DMA,
        ],
    )
    def seg_kernel(
        g_hbm, row_hbm, col_hbm, zeros_hbm, out_hbm,
        cidx_st, ridx_st, rows2, sacc, sg0, sg1, sr, sc2,
    ):
        semg = (sg0, sg1)
        c = lax.axis_index("c")
        s = lax.axis_index("s")
        nch = jnp.where(c == SC_SLOW_CORE, nch0, nch1)
        nblk = nch // NBC
        base = jnp.where(c == SC_SLOW_CORE, 0, a_ch) + s * nch
        for j in range(RPS // CHUNK):
            pltpu.sync_copy(
                zeros_hbm, sacc.at[pl.ds(s * RPS + j * CHUNK, CHUNK)]
            )
        pltpu.sync_copy(col_hbm.at[pl.ds(base, NBC)], cidx_st.at[0])
        pltpu.sync_copy(row_hbm.at[pl.ds(base, NBC)], ridx_st.at[0])
        plsc.subcore_barrier()
        pltpu.async_copy(g_hbm.at[cidx_st.at[0, 0]], rows2.at[0], sg0)
        pltpu.async_copy(g_hbm.at[cidx_st.at[0, 1]], rows2.at[1], sg1)

        def blk(k, _):
            t = lax.rem(k, 2)
            tn = lax.rem(k + 1, 2)
            pltpu.async_copy(
                row_hbm.at[pl.ds(base + (k + 1) * NBC, NBC)],
                ridx_st.at[tn],
                sr,
            )
            pltpu.async_copy(
                col_hbm.at[pl.ds(base + (k + 1) * NBC, NBC)],
                cidx_st.at[tn],
                sc2,
            )
            for jj in range(NBC):
                b = jj % 2
                if jj == NBC - 2:
                    pltpu.make_async_copy(
                        col_hbm.at[pl.ds(base + (k + 1) * NBC, NBC)],
                        cidx_st.at[tn],
                        sc2,
                    ).wait()
                if jj < NBC - 2:
                    nxt = cidx_st.at[t, jj + 2]
                else:
                    nxt = cidx_st.at[tn, jj + 2 - NBC]
                pltpu.make_async_copy(
                    g_hbm.at[cidx_st.at[t, jj]], rows2.at[b], semg[b]
                ).wait()
                pltpu.sync_copy(rows2.at[b], sacc.at[ridx_st.at[t, jj]], add=True)
                pltpu.async_copy(g_hbm.at[nxt], rows2.at[b], semg[b])
            pltpu.make_async_copy(
                row_hbm.at[pl.ds(base + (k + 1) * NBC, NBC)],
                ridx_st.at[tn],
                sr,
            ).wait()
            return 0

        lax.fori_loop(0, nblk, blk, 0)
        tl = lax.rem(nblk, 2)
        pltpu.make_async_copy(g_hbm.at[cidx_st.at[tl, 0]], rows2.at[0], sg0).wait()
        pltpu.make_async_copy(g_hbm.at[cidx_st.at[tl, 1]], rows2.at[1], sg1).wait()
        plsc.subcore_barrier()
        pltpu.sync_copy(
            sacc.at[pl.ds(s * RPS, RPS)], out_hbm.at[c, pl.ds(s * RPS, RPS)]
        )

    return seg_kernel


def _tc_g_body(x_ref, w_ref, db_ref, g_ref):
    db = db_ref[...]
    deg = 1.0 + db[:, 0] + db[:, 1]
    d = lax.rsqrt(deg)
    h = jnp.dot(x_ref[...], w_ref[...], preferred_element_type=jnp.float32)
    g_ref[...] = h * d[:, None]


_tc_g = pl.pallas_call(
    _tc_g_body,
    grid=(N_PAD // MB,),
    in_specs=[
        pl.BlockSpec((MB, D), lambda i: (i, 0)),
        pl.BlockSpec((D, D), lambda i: (0, 0)),
        pl.BlockSpec((MB, NC), lambda i: (i, 0)),
    ],
    out_specs=pl.BlockSpec((MB, D), lambda i: (i, 0)),
    out_shape=jax.ShapeDtypeStruct((N_PAD, D), jnp.float32),
)


def _tc_out_body(s_ref, g_ref, db_ref, b_ref, o_ref):
    db = db_ref[...]
    deg = 1.0 + db[:, 0] + db[:, 1]
    d = lax.rsqrt(deg)
    sv = s_ref[...]
    tot = sv[0] + sv[1] + g_ref[...]
    o_ref[...] = tot * d[:, None] + b_ref[...]


def _tc_out(n_nodes, ob):
    return pl.pallas_call(
        _tc_out_body,
        grid=(n_nodes // ob,),
        in_specs=[
            pl.BlockSpec((NC, ob, D), lambda i: (0, i, 0)),
            pl.BlockSpec((ob, D), lambda i: (i, 0)),
            pl.BlockSpec((ob, NC), lambda i: (i, 0)),
            pl.BlockSpec((1, D), lambda i: (0, 0)),
        ],
        out_specs=pl.BlockSpec((ob, D), lambda i: (i, 0)),
        out_shape=jax.ShapeDtypeStruct((n_nodes, D), jnp.float32),
    )


@jax.jit
def kernel(x, edge_index, weight, bias):
    n = x.shape[0]
    e = edge_index.shape[1]
    row = edge_index[0].astype(jnp.int32)
    col = edge_index[1].astype(jnp.int32)
    epb = NS * CHUNK * NBC * 4
    e_pad = ((e + epb - 1) // epb) * epb
    padv = jnp.full((e_pad - e + 8 * CHUNK,), n, jnp.int32)
    row2d = jnp.concatenate([row, padv]).reshape(-1, CHUNK)
    col2d = jnp.concatenate([col, padv]).reshape(-1, CHUNK)
    x_p = jnp.zeros((N_PAD, x.shape[1]), jnp.float32).at[:n, :].set(x)

    zerosd = jnp.zeros((CHUNK, D), jnp.float32)

    degbuf = _sc_degree(e_pad)(row2d)
    db = jnp.transpose(degbuf.reshape(NC, N_PAD))
    g = _tc_g(x_p, weight, db)
    s = _sc_segsum(e_pad)(g, row2d, col2d, zerosd)
    out = _tc_out(n, 2000)(s, g, db, bias.reshape(1, D))
    return out


# speedup vs baseline: 1.1083x; 1.1083x over previous
import functools

import jax
import jax.numpy as jnp
from jax import lax
from jax.experimental import pallas as pl
from jax.experimental.pallas import tpu as pltpu
from jax.experimental.pallas import tpu_sc as plsc

NC = 2
NS = 16
L = 16
NW = NC * NS

D = 128
CHUNK = 128
NBC = 8
MB = 1280
N_PAD = 10240
RPS = N_PAD // NS


def _mesh():
    return plsc.VectorSubcoreMesh(
        core_axis_name="c", subcore_axis_name="s", num_cores=NC, num_subcores=NS
    )


@functools.lru_cache(maxsize=None)
def _sc_degree(e_pad):
    ept = e_pad // NW
    nch = ept // CHUNK

    @functools.partial(
        pl.kernel,
        out_type=jax.ShapeDtypeStruct((NC * N_PAD,), jnp.float32),
        mesh=_mesh(),
        scratch_types=[
            pltpu.VMEM((ept // CHUNK, CHUNK), jnp.int32),
            pltpu.VMEM((CHUNK,), jnp.float32),
            pltpu.VMEM((RPS,), jnp.float32),
            pltpu.VMEM_SHARED((N_PAD,), jnp.float32),
            pltpu.SemaphoreType.DMA,
            pltpu.SemaphoreType.DMA,
            pltpu.SemaphoreType.DMA,
            pltpu.SemaphoreType.DMA,
        ],
    )
    def deg_kernel(row_hbm, out_hbm, idx_a, ones_v, zero_v, s1, sm0, sm1, sm2, sm3):
        c = lax.axis_index("c")
        s = lax.axis_index("s")
        wid = s * NC + c
        sems = (sm0, sm1, sm2, sm3)
        ones = jnp.ones((L,), jnp.float32)
        zeros = jnp.zeros((L,), jnp.float32)
        for j in range(CHUNK // L):
            ones_v[pl.ds(j * L, L)] = ones

        def zfill(i, _):
            zero_v[pl.ds(i * L, L)] = zeros
            return 0

        lax.fori_loop(0, RPS // L, zfill, 0)
        pltpu.sync_copy(zero_v, s1.at[pl.ds(s * RPS, RPS)])
        pltpu.sync_copy(row_hbm.at[pl.ds(wid * nch, nch)], idx_a)
        plsc.subcore_barrier()

        def quad(i, _):
            for b in range(4):
                j = 4 * i + b
                pltpu.make_async_copy(ones_v, s1.at[idx_a.at[j]], sems[b]).wait()
                pltpu.async_copy(ones_v, s1.at[idx_a.at[j + 4]], sems[b], add=True)
            return 0

        for b in range(4):
            pltpu.async_copy(ones_v, s1.at[idx_a.at[b]], sems[b], add=True)
        lax.fori_loop(0, nch // 4 - 1, quad, 0)
        for b in range(4):
            pltpu.make_async_copy(ones_v, s1.at[idx_a.at[0]], sems[b]).wait()
        plsc.subcore_barrier()
        pltpu.sync_copy(
            s1.at[pl.ds(s * RPS, RPS)],
            out_hbm.at[pl.ds(c * N_PAD + s * RPS, RPS)],
        )

    return deg_kernel


@functools.lru_cache(maxsize=None)
def _sc_segsum(e_pad):
    ept = e_pad // NW
    nch = ept // CHUNK
    nblk = nch // NBC

    @functools.partial(
        pl.kernel,
        out_type=jax.ShapeDtypeStruct((NC, N_PAD, D), jnp.float32),
        mesh=_mesh(),
        scratch_types=[
            pltpu.VMEM((nch + 8, CHUNK), jnp.int32),
            pltpu.VMEM((2, NBC, CHUNK), jnp.int32),
            pltpu.VMEM((2, CHUNK, D), jnp.float32),
            pltpu.VMEM_SHARED((N_PAD, D), jnp.float32),
            pltpu.SemaphoreType.DMA,
            pltpu.SemaphoreType.DMA,
            pltpu.SemaphoreType.DMA,
        ],
    )
    def seg_kernel(
        g_hbm, row_hbm, col_hbm, zeros_hbm, out_hbm,
        cidx_a, ridx_st, rows2, sacc, sg0, sg1, sr,
    ):
        semg = (sg0, sg1)
        c = lax.axis_index("c")
        s = lax.axis_index("s")
        wid = s * NC + c
        for j in range(RPS // CHUNK):
            pltpu.sync_copy(
                zeros_hbm, sacc.at[pl.ds(s * RPS + j * CHUNK, CHUNK)]
            )
        pltpu.sync_copy(col_hbm.at[pl.ds(wid * nch, nch + 8)], cidx_a)
        pltpu.sync_copy(row_hbm.at[pl.ds(wid * nch, NBC)], ridx_st.at[0])
        plsc.subcore_barrier()
        pltpu.async_copy(g_hbm.at[cidx_a.at[0]], rows2.at[0], sg0)
        pltpu.async_copy(g_hbm.at[cidx_a.at[1]], rows2.at[1], sg1)

        def blk(k, _):
            t = lax.rem(k, 2)
            tn = lax.rem(k + 1, 2)
            pltpu.async_copy(
                row_hbm.at[pl.ds(wid * nch + (k + 1) * NBC, NBC)],
                ridx_st.at[tn],
                sr,
            )
            for jj in range(NBC):
                b = jj % 2
                j = k * NBC + jj
                pltpu.make_async_copy(
                    g_hbm.at[cidx_a.at[j]], rows2.at[b], semg[b]
                ).wait()
                pltpu.sync_copy(rows2.at[b], sacc.at[ridx_st.at[t, jj]], add=True)
                pltpu.async_copy(g_hbm.at[cidx_a.at[j + 2]], rows2.at[b], semg[b])
            pltpu.make_async_copy(
                row_hbm.at[pl.ds(wid * nch + (k + 1) * NBC, NBC)],
                ridx_st.at[tn],
                sr,
            ).wait()
            return 0

        lax.fori_loop(0, nblk, blk, 0)
        pltpu.make_async_copy(g_hbm.at[cidx_a.at[nch]], rows2.at[0], sg0).wait()
        pltpu.make_async_copy(g_hbm.at[cidx_a.at[nch + 1]], rows2.at[1], sg1).wait()
        plsc.subcore_barrier()
        pltpu.sync_copy(
            sacc.at[pl.ds(s * RPS, RPS)], out_hbm.at[c, pl.ds(s * RPS, RPS)]
        )

    return seg_kernel


def _tc_g_body(x_ref, w_ref, db_ref, g_ref):
    db = db_ref[...]
    deg = 1.0 + db[:, 0] + db[:, 1]
    d = lax.rsqrt(deg)
    h = jnp.dot(x_ref[...], w_ref[...], preferred_element_type=jnp.float32)
    g_ref[...] = h * d[:, None]


_tc_g = pl.pallas_call(
    _tc_g_body,
    grid=(N_PAD // MB,),
    in_specs=[
        pl.BlockSpec((MB, D), lambda i: (i, 0)),
        pl.BlockSpec((D, D), lambda i: (0, 0)),
        pl.BlockSpec((MB, NC), lambda i: (i, 0)),
    ],
    out_specs=pl.BlockSpec((MB, D), lambda i: (i, 0)),
    out_shape=jax.ShapeDtypeStruct((N_PAD, D), jnp.float32),
)


def _tc_out_body(s_ref, g_ref, db_ref, b_ref, o_ref):
    db = db_ref[...]
    deg = 1.0 + db[:, 0] + db[:, 1]
    d = lax.rsqrt(deg)
    sv = s_ref[...]
    tot = sv[0] + sv[1] + g_ref[...]
    o_ref[...] = tot * d[:, None] + b_ref[...]


def _tc_out(n_nodes, ob):
    return pl.pallas_call(
        _tc_out_body,
        grid=(n_nodes // ob,),
        in_specs=[
            pl.BlockSpec((NC, ob, D), lambda i: (0, i, 0)),
            pl.BlockSpec((ob, D), lambda i: (i, 0)),
            pl.BlockSpec((ob, NC), lambda i: (i, 0)),
            pl.BlockSpec((1, D), lambda i: (0, 0)),
        ],
        out_specs=pl.BlockSpec((ob, D), lambda i: (i, 0)),
        out_shape=jax.ShapeDtypeStruct((n_nodes, D), jnp.float32),
    )


@jax.jit
def kernel(x, edge_index, weight, bias):
    n = x.shape[0]
    e = edge_index.shape[1]
    row = edge_index[0].astype(jnp.int32)
    col = edge_index[1].astype(jnp.int32)
    epb = NW * CHUNK * NBC
    e_pad = ((e + epb - 1) // epb) * epb
    padv = jnp.full((e_pad - e + 8 * CHUNK,), n, jnp.int32)
    row2d = jnp.concatenate([row, padv]).reshape(-1, CHUNK)
    col2d = jnp.concatenate([col, padv]).reshape(-1, CHUNK)
    x_p = jnp.zeros((N_PAD, x.shape[1]), jnp.float32).at[:n, :].set(x)

    zerosd = jnp.zeros((CHUNK, D), jnp.float32)

    degbuf = _sc_degree(e_pad)(row2d)
    db = jnp.transpose(degbuf.reshape(NC, N_PAD))
    g = _tc_g(x_p, weight, db)
    s = _sc_segsum(e_pad)(g, row2d, col2d, zerosd)
    out = _tc_out(n, 2000)(s, g, db, bias.reshape(1, D))
    return out
